# baseline (speedup 1.0000x reference)
import functools

import jax
import jax.numpy as jnp
from jax import lax
from jax.experimental import pallas as pl
from jax.experimental.pallas import tpu as pltpu
from jax.experimental.pallas import tpu_sc as plsc

D_EMB = 16
D_HID = 768
BATCH = 1024
SEQ = 50
NTOK = BATCH * SEQ
NBLK_OUT = NTOK // 8

_info = plsc.get_sparse_core_info()
_NC, _NS = _info.num_cores, _info.num_subcores
_NW = _NC * _NS
_B_PER_W = NTOK // _NW
_CH = 320
_NCH = _B_PER_W // _CH
_G = 16
_NGRP = _CH // _G

_mesh = plsc.VectorSubcoreMesh(core_axis_name="c", subcore_axis_name="s")


@functools.partial(
    pl.kernel,
    out_type=jax.ShapeDtypeStruct((NBLK_OUT, 8, D_EMB), jnp.float32),
    mesh=_mesh,
    scratch_types=[
        pltpu.VMEM((_B_PER_W + _G,), jnp.int32),
        pltpu.VMEM((_G, 8, D_EMB), jnp.float32),
        pltpu.VMEM((_CH // 8, 8, D_EMB), jnp.float32),
        pltpu.SemaphoreType.DMA((_G,)),
    ],
    compiler_params=pltpu.CompilerParams(needs_layout_passes=False),
)
def _sc_gather(table_hbm, idx_hbm, out_hbm, idx_v, ring_v, rows_v, sems):
    wid = lax.axis_index("s") * _NC + lax.axis_index("c")
    base = wid * _B_PER_W
    pltpu.sync_copy(
        idx_hbm.at[pl.ds(base, _B_PER_W)], idx_v.at[pl.ds(0, _B_PER_W)]
    )
    iota = lax.iota(jnp.int32, _G)
    mask7 = jnp.int32(~7)

    for c in range(_NCH):
        cbase = c * _CH
        prow = idx_v[pl.ds(cbase, _G)] & mask7
        for b in range(_G):
            pltpu.async_copy(
                table_hbm.at[pl.ds(pl.multiple_of(prow[b], 8), 8)], ring_v.at[b], sems.at[b]
            )

        def _body(i, carry, cbase=cbase):
            tloc = i * _G
            lo = idx_v[pl.ds(cbase + tloc, _G)] & jnp.int32(7)
            nrow = idx_v[pl.ds(cbase + tloc + _G, _G)] & mask7
            tvec = tloc + iota
            br = lax.shift_right_logical(tvec, jnp.int32(3))
            sub = tvec & jnp.int32(7)
            for b in range(_G):
                pltpu.make_async_copy(
                    table_hbm.at[pl.ds(0, 8)], ring_v.at[b], sems.at[b]
                ).wait()
            for col in range(D_EMB):
                cv = jnp.full((_G,), col, jnp.int32)
                vals = plsc.load_gather(ring_v, [iota, lo, cv])
                plsc.store_scatter(rows_v, [br, sub, cv], vals)

            @pl.when(i + 1 < _NGRP)
            def _issue():
                for b in range(_G):
                    pltpu.async_copy(
                        table_hbm.at[pl.ds(pl.multiple_of(nrow[b], 8), 8)],
                        ring_v.at[b],
                        sems.at[b],
                    )

            return carry

        lax.fori_loop(0, _NGRP, _body, 0)
        pltpu.sync_copy(
            rows_v, out_hbm.at[pl.ds((base + cbase) // 8, _CH // 8)]
        )


_BROW = 64


def _proj_body(emb_ref, w_ref, b_ref, out_ref):
    e = emb_ref[...].reshape(_BROW * SEQ, D_EMB)
    h = (
        jnp.dot(e, w_ref[...], preferred_element_type=jnp.float32)
        + b_ref[...].reshape(1, D_HID)
    )
    out_ref[...] = h.reshape(_BROW, SEQ, D_HID)


def _project(emb3, W, b3):
    grid = (BATCH // _BROW,)
    nblk = _BROW * SEQ // 8
    return pl.pallas_call(
        _proj_body,
        grid=grid,
        in_specs=[
            pl.BlockSpec((nblk, 8, D_EMB), lambda i: (i, 0, 0)),
            pl.BlockSpec((D_EMB, D_HID), lambda i: (0, 0)),
            pl.BlockSpec((1, 1, D_HID), lambda i: (0, 0, 0)),
        ],
        out_specs=pl.BlockSpec((_BROW, SEQ, D_HID), lambda i: (i, 0, 0)),
        out_shape=jax.ShapeDtypeStruct((BATCH, SEQ, D_HID), jnp.float32),
    )(emb3, W, b3)


def kernel(idxs, table, W, b):
    flat = idxs.reshape(-1)
    emb3 = _sc_gather(table, flat)
    return _project(emb3, W, b.reshape(1, 1, D_HID))
